# baseline (speedup 1.0000x reference)
import jax
import jax.numpy as jnp
from jax.experimental import pallas as pl

_NUM_PATCHES = 1024
_DIM = 768


_BB = 4


def _bcast_body(table_ref, out_ref):
    out_ref[...] = jnp.broadcast_to(table_ref[...][None], out_ref.shape)


def kernel(x, table):
    batch = x.shape[0]
    return pl.pallas_call(
        _bcast_body,
        grid=(batch // _BB,),
        in_specs=[pl.BlockSpec((_NUM_PATCHES, _DIM), lambda b: (0, 0))],
        out_specs=pl.BlockSpec((_BB, _NUM_PATCHES, _DIM), lambda b: (b, 0, 0)),
        out_shape=jax.ShapeDtypeStruct((batch, _NUM_PATCHES, _DIM), jnp.float32),
    )(table)
